# baseline (speedup 1.0000x reference)
import functools

import jax
import jax.numpy as jnp
from jax import lax
from jax.experimental import pallas as pl
from jax.experimental.pallas import tpu as pltpu
from jax.experimental.pallas import tpu_sc as plsc

T, D, H, E = 2048, 768, 2048, 8

BT = 256
NBLK = T // BT + E - 1
NPAD = NBLK * BT

TB = 256
NTB = T // TB

NC, NS = 2, 16
NW = NC * NS
SPW = NPAD // NW
TPW = T // NW

_f32 = jnp.float32
_i32 = jnp.int32


def _route_body(x_ref, gam_ref, bet_ref, wg_ref, bg_ref,
                h_ref, slot_ref, srow_ref, be_ref, bv_ref, aux_ref):
    x = x_ref[...]
    mu = jnp.mean(x, axis=1, keepdims=True)
    xc = x - mu
    var = jnp.mean(xc * xc, axis=1, keepdims=True)
    h = xc * lax.rsqrt(var + 1e-5) * gam_ref[...] + bet_ref[...]
    h_ref[...] = h

    logits = lax.dot_general(h, wg_ref[...], (((1,), (0,)), ((), ())),
                             preferred_element_type=_f32) + bg_ref[...]
    m = jnp.max(logits, axis=1, keepdims=True)
    p = jnp.exp(logits - m)
    probs = p / jnp.sum(p, axis=1, keepdims=True)

    e_iota = lax.broadcasted_iota(_i32, (T, E), 1)
    maxp = jnp.max(probs, axis=1, keepdims=True)
    idx = jnp.min(jnp.where(probs == maxp, e_iota, E), axis=1, keepdims=True)
    oh = (e_iota == idx).astype(_f32)

    counts = jnp.sum(oh, axis=0, keepdims=True)
    imp = jnp.sum(probs, axis=0, keepdims=True) * (1.0 / T)
    aux_ref[...] = E * jnp.sum(imp * counts * (1.0 / T), axis=(0, 1),
                               keepdims=True)

    ci = counts.astype(_i32)
    padded = ((ci + (BT - 1)) // BT) * BT
    er = lax.broadcasted_iota(_i32, (E, E), 0)
    ec = lax.broadcasted_iota(_i32, (E, E), 1)
    strict_lt = (er < ec).astype(_f32)
    base_f = lax.dot_general(padded.astype(_f32), strict_lt,
                             (((1,), (0,)), ((), ())),
                             preferred_element_type=_f32)

    tr = lax.broadcasted_iota(_i32, (TB, TB), 0)
    tc = lax.broadcasted_iota(_i32, (TB, TB), 1)
    tril = (tr >= tc).astype(_f32)
    eye = (tr == tc).astype(_f32)

    running = jnp.zeros((1, E), _f32)
    slot_t_parts = []
    for kb in range(NTB):
        ohb = oh[kb * TB:(kb + 1) * TB]
        incl = lax.dot_general(tril, ohb, (((1,), (0,)), ((), ())),
                               preferred_element_type=_f32)
        excl = incl - ohb + running
        slot_b = jnp.sum(ohb * (base_f + excl), axis=1, keepdims=True)
        slot_ref[kb * TB:(kb + 1) * TB, :] = slot_b.astype(_i32)
        slot_t_parts.append(lax.dot_general(slot_b, eye,
                                            (((0,), (0,)), ((), ())),
                                            preferred_element_type=_f32))
        running = running + jnp.sum(ohb, axis=0, keepdims=True)
    srow_ref[...] = jnp.concatenate(slot_t_parts, axis=1).astype(_i32)

    base_i = base_f.astype(_i32)
    starts = base_i // BT
    ends = (base_i + padded) // BT
    b8 = lax.broadcasted_iota(_i32, (NBLK, E), 0)
    e8 = lax.broadcasted_iota(_i32, (NBLK, E), 1)
    inb = (b8 >= starts) & (b8 < ends)
    be = jnp.sum(jnp.where(inb, e8, 0), axis=1, keepdims=True)
    used = jnp.sum(padded) // BT
    maxe = jnp.max(jnp.where(ci > 0, lax.broadcasted_iota(_i32, (1, E), 1), 0))
    bcol = lax.broadcasted_iota(_i32, (NBLK, 1), 0)
    valid = bcol < used
    be_ref[...] = jnp.where(valid, be, maxe)
    bv_ref[...] = valid.astype(_i32)


_route = pl.pallas_call(
    _route_body,
    out_shape=[
        jax.ShapeDtypeStruct((T, D), _f32),
        jax.ShapeDtypeStruct((T, 1), _i32),
        jax.ShapeDtypeStruct((1, T), _i32),
        jax.ShapeDtypeStruct((NBLK, 1), _i32),
        jax.ShapeDtypeStruct((NBLK, 1), _i32),
        jax.ShapeDtypeStruct((1, 1), _f32),
    ],
)


def _ffn_body(be_s, bv_s, h_ref, srow_ref, w1_ref, b1_ref, w2_ref, b2_ref,
              ys_ref):
    b = pl.program_id(0)

    @pl.when(bv_s[b] == 1)
    def _():
        bf16 = jnp.bfloat16
        s_col = lax.broadcasted_iota(_i32, (BT, 1), 0) + b * BT
        p = (srow_ref[...] == s_col).astype(bf16)
        rows = lax.dot_general(p, h_ref[...].astype(bf16),
                               (((1,), (0,)), ((), ())),
                               preferred_element_type=_f32)
        a = lax.dot_general(rows, w1_ref[0], (((1,), (0,)), ((), ())),
                            preferred_element_type=_f32) + b1_ref[0]
        a = 0.5 * a * (1.0 + lax.erf(a * 0.7071067811865476))
        y = lax.dot_general(a, w2_ref[0], (((1,), (0,)), ((), ())),
                            preferred_element_type=_f32) + b2_ref[0]
        ys_ref[...] = y


_ffn = pl.pallas_call(
    _ffn_body,
    grid_spec=pltpu.PrefetchScalarGridSpec(
        num_scalar_prefetch=2,
        grid=(NBLK,),
        in_specs=[
            pl.BlockSpec((T, D), lambda b, be, bv: (0, 0)),
            pl.BlockSpec((1, T), lambda b, be, bv: (0, 0)),
            pl.BlockSpec((1, D, H), lambda b, be, bv: (be[b], 0, 0)),
            pl.BlockSpec((1, 1, H), lambda b, be, bv: (be[b], 0, 0)),
            pl.BlockSpec((1, H, D), lambda b, be, bv: (be[b], 0, 0)),
            pl.BlockSpec((1, 1, D), lambda b, be, bv: (be[b], 0, 0)),
        ],
        out_specs=pl.BlockSpec((BT, D), lambda b, be, bv: (b, 0)),
    ),
    out_shape=jax.ShapeDtypeStruct((NPAD, D), _f32),
)


def _combine_body(ys_hbm, slot_hbm, x_hbm, out_hbm, idx_v, rows_v, x_v, sem):
    wid = lax.axis_index("s") * NC + lax.axis_index("c")
    base = wid * TPW
    pltpu.sync_copy(slot_hbm.at[pl.ds(base, TPW)], idx_v)
    pltpu.async_copy(ys_hbm.at[idx_v], rows_v, sem).wait()
    pltpu.sync_copy(x_hbm.at[pl.ds(base, TPW)], x_v)

    def row(i, carry):
        for j in range(D // 16):
            sl = pl.ds(j * 16, 16)
            x_v[i, sl] = x_v[i, sl] + rows_v[i, sl]
        return carry

    lax.fori_loop(0, TPW, row, 0)
    pltpu.sync_copy(x_v, out_hbm.at[pl.ds(base, TPW)])


@functools.cache
def _make_combine():
    return pl.kernel(
        _combine_body,
        out_type=jax.ShapeDtypeStruct((T, D), _f32),
        mesh=plsc.VectorSubcoreMesh(core_axis_name="c", subcore_axis_name="s"),
        scratch_types=[
            pltpu.VMEM((TPW,), _i32),
            pltpu.VMEM((TPW, D), _f32),
            pltpu.VMEM((TPW, D), _f32),
            pltpu.SemaphoreType.DMA,
        ],
    )


def kernel(x, gamma, beta, Wg, bg, W1, b1, W2, b2):
    h, slot2, srow, be2, bv2, aux = _route(
        x, gamma.reshape(1, D), beta.reshape(1, D), Wg, bg.reshape(1, E))
    ys = _ffn(be2.reshape(NBLK), bv2.reshape(NBLK), h, srow,
              W1, b1.reshape(E, 1, H), W2, b2.reshape(E, 1, D))
    out = _make_combine()(ys, slot2.reshape(T), x)
    return out, aux[0, 0]


# speedup vs baseline: 1.1057x; 1.1057x over previous
import functools

import jax
import jax.numpy as jnp
from jax import lax
from jax.experimental import pallas as pl
from jax.experimental.pallas import tpu as pltpu
from jax.experimental.pallas import tpu_sc as plsc

T, D, H, E = 2048, 768, 2048, 8

BT = 256
NBLK = T // BT + E - 1
NPAD = NBLK * BT

TB = 256
NTB = T // TB

NC, NS = 2, 16
NW = NC * NS
SPW = NPAD // NW
TPW = T // NW

_f32 = jnp.float32
_i32 = jnp.int32


def _route_body(x_ref, gam_ref, bet_ref, wg_ref, bg_ref,
                h_ref, slot_ref, be_ref, bv_ref, aux_ref):
    x = x_ref[...]
    mu = jnp.mean(x, axis=1, keepdims=True)
    xc = x - mu
    var = jnp.mean(xc * xc, axis=1, keepdims=True)
    h = xc * lax.rsqrt(var + 1e-5) * gam_ref[...] + bet_ref[...]
    h_ref[...] = h

    logits = lax.dot_general(h, wg_ref[...], (((1,), (0,)), ((), ())),
                             preferred_element_type=_f32) + bg_ref[...]
    m = jnp.max(logits, axis=1, keepdims=True)
    p = jnp.exp(logits - m)
    probs = p / jnp.sum(p, axis=1, keepdims=True)

    e_iota = lax.broadcasted_iota(_i32, (T, E), 1)
    maxp = jnp.max(probs, axis=1, keepdims=True)
    idx = jnp.min(jnp.where(probs == maxp, e_iota, E), axis=1, keepdims=True)
    oh = (e_iota == idx).astype(_f32)

    counts = jnp.sum(oh, axis=0, keepdims=True)
    imp = jnp.sum(probs, axis=0, keepdims=True) * (1.0 / T)
    aux_ref[...] = E * jnp.sum(imp * counts * (1.0 / T), axis=(0, 1),
                               keepdims=True)

    ci = counts.astype(_i32)
    padded = ((ci + (BT - 1)) // BT) * BT
    er = lax.broadcasted_iota(_i32, (E, E), 0)
    ec = lax.broadcasted_iota(_i32, (E, E), 1)
    strict_lt = (er < ec).astype(_f32)
    base_f = lax.dot_general(padded.astype(_f32), strict_lt,
                             (((1,), (0,)), ((), ())),
                             preferred_element_type=_f32)

    tr = lax.broadcasted_iota(_i32, (TB, TB), 0)
    tc = lax.broadcasted_iota(_i32, (TB, TB), 1)
    tril = (tr >= tc).astype(_f32)

    running = jnp.zeros((1, E), _f32)
    for kb in range(NTB):
        ohb = oh[kb * TB:(kb + 1) * TB]
        incl = lax.dot_general(tril, ohb, (((1,), (0,)), ((), ())),
                               preferred_element_type=_f32)
        excl = incl - ohb + running
        slot_b = jnp.sum(ohb * (base_f + excl), axis=1, keepdims=True)
        slot_ref[kb * TB:(kb + 1) * TB, :] = slot_b.astype(_i32)
        running = running + jnp.sum(ohb, axis=0, keepdims=True)

    base_i = base_f.astype(_i32)
    starts = base_i // BT
    ends = (base_i + padded) // BT
    b8 = lax.broadcasted_iota(_i32, (NBLK, E), 0)
    e8 = lax.broadcasted_iota(_i32, (NBLK, E), 1)
    inb = (b8 >= starts) & (b8 < ends)
    be = jnp.sum(jnp.where(inb, e8, 0), axis=1, keepdims=True)
    used = jnp.sum(padded) // BT
    maxe = jnp.max(jnp.where(ci > 0, lax.broadcasted_iota(_i32, (1, E), 1), 0))
    bcol = lax.broadcasted_iota(_i32, (NBLK, 1), 0)
    valid = bcol < used
    be_ref[...] = jnp.where(valid, be, maxe)
    bv_ref[...] = valid.astype(_i32)


_route = pl.pallas_call(
    _route_body,
    out_shape=[
        jax.ShapeDtypeStruct((T, D), _f32),
        jax.ShapeDtypeStruct((T, 1), _i32),
        jax.ShapeDtypeStruct((NBLK, 1), _i32),
        jax.ShapeDtypeStruct((NBLK, 1), _i32),
        jax.ShapeDtypeStruct((1, 1), _f32),
    ],
)


def _scatter_body(h_hbm, slot_hbm, hs_hbm, idx_v, rows_v, sem):
    wid = lax.axis_index("s") * NC + lax.axis_index("c")
    base = wid * TPW
    pltpu.sync_copy(slot_hbm.at[pl.ds(base, TPW)], idx_v)
    pltpu.sync_copy(h_hbm.at[pl.ds(base, TPW)], rows_v)
    pltpu.async_copy(rows_v, hs_hbm.at[idx_v], sem).wait()


@functools.cache
def _make_scatter():
    return pl.kernel(
        _scatter_body,
        out_type=jax.ShapeDtypeStruct((NPAD, D), _f32),
        mesh=plsc.VectorSubcoreMesh(core_axis_name="c", subcore_axis_name="s"),
        scratch_types=[
            pltpu.VMEM((TPW,), _i32),
            pltpu.VMEM((TPW, D), _f32),
            pltpu.SemaphoreType.DMA,
        ],
    )


def _ffn_body(be_s, bv_s, hs_ref, w1_ref, b1_ref, w2_ref, b2_ref, ys_ref):
    b = pl.program_id(0)

    @pl.when(bv_s[b] == 1)
    def _():
        a = lax.dot_general(hs_ref[...], w1_ref[0], (((1,), (0,)), ((), ())),
                            preferred_element_type=_f32) + b1_ref[0]
        a = 0.5 * a * (1.0 + lax.erf(a * 0.7071067811865476))
        y = lax.dot_general(a, w2_ref[0], (((1,), (0,)), ((), ())),
                            preferred_element_type=_f32) + b2_ref[0]
        ys_ref[...] = y


_ffn = pl.pallas_call(
    _ffn_body,
    grid_spec=pltpu.PrefetchScalarGridSpec(
        num_scalar_prefetch=2,
        grid=(NBLK,),
        in_specs=[
            pl.BlockSpec((BT, D), lambda b, be, bv: (b, 0)),
            pl.BlockSpec((1, D, H), lambda b, be, bv: (be[b], 0, 0)),
            pl.BlockSpec((1, 1, H), lambda b, be, bv: (be[b], 0, 0)),
            pl.BlockSpec((1, H, D), lambda b, be, bv: (be[b], 0, 0)),
            pl.BlockSpec((1, 1, D), lambda b, be, bv: (be[b], 0, 0)),
        ],
        out_specs=pl.BlockSpec((BT, D), lambda b, be, bv: (b, 0)),
    ),
    out_shape=jax.ShapeDtypeStruct((NPAD, D), _f32),
)


def _combine_body(ys_hbm, slot_hbm, x_hbm, out_hbm, idx_v, rows_v, x_v, sem):
    wid = lax.axis_index("s") * NC + lax.axis_index("c")
    base = wid * TPW
    pltpu.sync_copy(slot_hbm.at[pl.ds(base, TPW)], idx_v)
    pltpu.async_copy(ys_hbm.at[idx_v], rows_v, sem).wait()
    pltpu.sync_copy(x_hbm.at[pl.ds(base, TPW)], x_v)

    def row(i, carry):
        for j in range(D // 16):
            sl = pl.ds(j * 16, 16)
            x_v[i, sl] = x_v[i, sl] + rows_v[i, sl]
        return carry

    lax.fori_loop(0, TPW, row, 0)
    pltpu.sync_copy(x_v, out_hbm.at[pl.ds(base, TPW)])


@functools.cache
def _make_combine():
    return pl.kernel(
        _combine_body,
        out_type=jax.ShapeDtypeStruct((T, D), _f32),
        mesh=plsc.VectorSubcoreMesh(core_axis_name="c", subcore_axis_name="s"),
        scratch_types=[
            pltpu.VMEM((TPW,), _i32),
            pltpu.VMEM((TPW, D), _f32),
            pltpu.VMEM((TPW, D), _f32),
            pltpu.SemaphoreType.DMA,
        ],
    )


def kernel(x, gamma, beta, Wg, bg, W1, b1, W2, b2):
    h, slot2, be2, bv2, aux = _route(
        x, gamma.reshape(1, D), beta.reshape(1, D), Wg, bg.reshape(1, E))
    hs = _make_scatter()(h, slot2.reshape(T))
    ys = _ffn(be2.reshape(NBLK), bv2.reshape(NBLK), hs,
              W1, b1.reshape(E, 1, H), W2, b2.reshape(E, 1, D))
    out = _make_combine()(ys, slot2.reshape(T), x)
    return out, aux[0, 0]


# speedup vs baseline: 1.1288x; 1.0209x over previous
import functools

import jax
import jax.numpy as jnp
from jax import lax
from jax.experimental import pallas as pl
from jax.experimental.pallas import tpu as pltpu
from jax.experimental.pallas import tpu_sc as plsc

T, D, H, E = 2048, 768, 2048, 8

BT = 256
NBLK = T // BT + E - 1
NPAD = NBLK * BT

TB = 256
NTB = T // TB

NC, NS = 2, 16
NW = NC * NS
SPW = NPAD // NW
TPW = T // NW

_f32 = jnp.float32
_i32 = jnp.int32


def _route_body(x_ref, gam_ref, bet_ref, wg_ref, bg_ref,
                h_ref, slot_ref, be_ref, bv_ref, aux_ref):
    x = x_ref[...]
    mu = jnp.mean(x, axis=1, keepdims=True)
    xc = x - mu
    var = jnp.mean(xc * xc, axis=1, keepdims=True)
    h = xc * lax.rsqrt(var + 1e-5) * gam_ref[...] + bet_ref[...]
    hbits = lax.bitcast_convert_type(h, _i32)

    def _rne16(bits):
        odd = lax.shift_right_logical(bits, 16) & 1
        return lax.shift_right_logical(bits + 0x7FFF + odd, 16)

    lo = _rne16(hbits[:, :D // 2])
    hi = _rne16(hbits[:, D // 2:])
    h_ref[...] = lax.shift_left(hi, 16) | lo

    logits = lax.dot_general(h, wg_ref[...], (((1,), (0,)), ((), ())),
                             preferred_element_type=_f32) + bg_ref[...]
    m = jnp.max(logits, axis=1, keepdims=True)
    p = jnp.exp(logits - m)
    probs = p / jnp.sum(p, axis=1, keepdims=True)

    e_iota = lax.broadcasted_iota(_i32, (T, E), 1)
    maxp = jnp.max(probs, axis=1, keepdims=True)
    idx = jnp.min(jnp.where(probs == maxp, e_iota, E), axis=1, keepdims=True)
    oh = (e_iota == idx).astype(_f32)

    counts = jnp.sum(oh, axis=0, keepdims=True)
    imp = jnp.sum(probs, axis=0, keepdims=True) * (1.0 / T)
    aux_ref[...] = E * jnp.sum(imp * counts * (1.0 / T), axis=(0, 1),
                               keepdims=True)

    ci = counts.astype(_i32)
    padded = ((ci + (BT - 1)) // BT) * BT
    er = lax.broadcasted_iota(_i32, (E, E), 0)
    ec = lax.broadcasted_iota(_i32, (E, E), 1)
    strict_lt = (er < ec).astype(_f32)
    base_f = lax.dot_general(padded.astype(_f32), strict_lt,
                             (((1,), (0,)), ((), ())),
                             preferred_element_type=_f32)

    tr = lax.broadcasted_iota(_i32, (TB, TB), 0)
    tc = lax.broadcasted_iota(_i32, (TB, TB), 1)
    tril = (tr >= tc).astype(_f32)

    running = jnp.zeros((1, E), _f32)
    for kb in range(NTB):
        ohb = oh[kb * TB:(kb + 1) * TB]
        incl = lax.dot_general(tril, ohb, (((1,), (0,)), ((), ())),
                               preferred_element_type=_f32)
        excl = incl - ohb + running
        slot_b = jnp.sum(ohb * (base_f + excl), axis=1, keepdims=True)
        slot_ref[kb * TB:(kb + 1) * TB, :] = slot_b.astype(_i32)
        running = running + jnp.sum(ohb, axis=0, keepdims=True)

    base_i = base_f.astype(_i32)
    starts = base_i // BT
    ends = (base_i + padded) // BT
    b8 = lax.broadcasted_iota(_i32, (NBLK, E), 0)
    e8 = lax.broadcasted_iota(_i32, (NBLK, E), 1)
    inb = (b8 >= starts) & (b8 < ends)
    be = jnp.sum(jnp.where(inb, e8, 0), axis=1, keepdims=True)
    used = jnp.sum(padded) // BT
    maxe = jnp.max(jnp.where(ci > 0, lax.broadcasted_iota(_i32, (1, E), 1), 0))
    bcol = lax.broadcasted_iota(_i32, (NBLK, 1), 0)
    valid = bcol < used
    be_ref[...] = jnp.where(valid, be, maxe)
    bv_ref[...] = valid.astype(_i32)


_route = pl.pallas_call(
    _route_body,
    out_shape=[
        jax.ShapeDtypeStruct((T, D // 2), _i32),
        jax.ShapeDtypeStruct((T, 1), _i32),
        jax.ShapeDtypeStruct((NBLK, 1), _i32),
        jax.ShapeDtypeStruct((NBLK, 1), _i32),
        jax.ShapeDtypeStruct((1, 1), _f32),
    ],
)


def _scatter_body(h_hbm, slot_hbm, hs_hbm, idx_v, rows_v, sem):
    wid = lax.axis_index("s") * NC + lax.axis_index("c")
    base = wid * TPW
    pltpu.sync_copy(slot_hbm.at[pl.ds(base, TPW)], idx_v)
    pltpu.sync_copy(h_hbm.at[pl.ds(base, TPW)], rows_v)
    pltpu.async_copy(rows_v, hs_hbm.at[idx_v], sem).wait()


@functools.cache
def _make_scatter():
    return pl.kernel(
        _scatter_body,
        out_type=jax.ShapeDtypeStruct((NPAD, D // 2), _i32),
        mesh=plsc.VectorSubcoreMesh(core_axis_name="c", subcore_axis_name="s"),
        scratch_types=[
            pltpu.VMEM((TPW,), _i32),
            pltpu.VMEM((TPW, D // 2), _i32),
            pltpu.SemaphoreType.DMA,
        ],
    )


def _ffn_body(be_s, bv_s, hs_ref, w1_ref, b1_ref, w2_ref, b2_ref, ys_ref):
    b = pl.program_id(0)

    @pl.when(bv_s[b] == 1)
    def _():
        pk = hs_ref[...]
        f_lo = lax.bitcast_convert_type(lax.shift_left(pk, 16), _f32)
        f_hi = lax.bitcast_convert_type(pk & jnp.int32(-65536), _f32)
        hs = jnp.concatenate([f_lo, f_hi], axis=1)
        a = lax.dot_general(hs, w1_ref[0], (((1,), (0,)), ((), ())),
                            preferred_element_type=_f32) + b1_ref[0]
        a = 0.5 * a * (1.0 + lax.erf(a * 0.7071067811865476))
        y = lax.dot_general(a, w2_ref[0], (((1,), (0,)), ((), ())),
                            preferred_element_type=_f32) + b2_ref[0]
        ys_ref[...] = y


_ffn = pl.pallas_call(
    _ffn_body,
    grid_spec=pltpu.PrefetchScalarGridSpec(
        num_scalar_prefetch=2,
        grid=(NBLK,),
        in_specs=[
            pl.BlockSpec((BT, D // 2), lambda b, be, bv: (b, 0)),
            pl.BlockSpec((1, D, H), lambda b, be, bv: (be[b], 0, 0)),
            pl.BlockSpec((1, 1, H), lambda b, be, bv: (be[b], 0, 0)),
            pl.BlockSpec((1, H, D), lambda b, be, bv: (be[b], 0, 0)),
            pl.BlockSpec((1, 1, D), lambda b, be, bv: (be[b], 0, 0)),
        ],
        out_specs=pl.BlockSpec((BT, D), lambda b, be, bv: (b, 0)),
    ),
    out_shape=jax.ShapeDtypeStruct((NPAD, D), _f32),
)


def _combine_body(ys_hbm, slot_hbm, x_hbm, out_hbm, idx_v, rows_v, x_v, sem):
    wid = lax.axis_index("s") * NC + lax.axis_index("c")
    base = wid * TPW
    pltpu.sync_copy(slot_hbm.at[pl.ds(base, TPW)], idx_v)
    pltpu.async_copy(ys_hbm.at[idx_v], rows_v, sem).wait()
    pltpu.sync_copy(x_hbm.at[pl.ds(base, TPW)], x_v)

    def row(i, carry):
        for j in range(D // 16):
            sl = pl.ds(j * 16, 16)
            x_v[i, sl] = x_v[i, sl] + rows_v[i, sl]
        return carry

    lax.fori_loop(0, TPW, row, 0)
    pltpu.sync_copy(x_v, out_hbm.at[pl.ds(base, TPW)])


@functools.cache
def _make_combine():
    return pl.kernel(
        _combine_body,
        out_type=jax.ShapeDtypeStruct((T, D), _f32),
        mesh=plsc.VectorSubcoreMesh(core_axis_name="c", subcore_axis_name="s"),
        scratch_types=[
            pltpu.VMEM((TPW,), _i32),
            pltpu.VMEM((TPW, D), _f32),
            pltpu.VMEM((TPW, D), _f32),
            pltpu.SemaphoreType.DMA,
        ],
    )


def kernel(x, gamma, beta, Wg, bg, W1, b1, W2, b2):
    h, slot2, be2, bv2, aux = _route(
        x, gamma.reshape(1, D), beta.reshape(1, D), Wg, bg.reshape(1, E))
    hs = _make_scatter()(h, slot2.reshape(T))
    ys = _ffn(be2.reshape(NBLK), bv2.reshape(NBLK), hs,
              W1, b1.reshape(E, 1, H), W2, b2.reshape(E, 1, D))
    out = _make_combine()(ys, slot2.reshape(T), x)
    return out, aux[0, 0]
